# baseline (speedup 1.0000x reference)
import functools

import jax
import jax.numpy as jnp
from jax import lax
from jax.experimental import pallas as pl
from jax.experimental.pallas import tpu as pltpu
from jax.experimental.pallas import tpu_sc as plsc

NUM_HEADS = 16
_NC, _NS = 2, 16
_NW = _NC * _NS
_CHUNK = 128


@functools.lru_cache(maxsize=None)
def _build(B):
    b_per_w = B // _NW
    n_chunks = b_per_w // _CHUNK
    mesh = plsc.VectorSubcoreMesh(core_axis_name="c", subcore_axis_name="s")

    @functools.partial(
        pl.kernel,
        mesh=mesh,
        out_type=jax.ShapeDtypeStruct((B, NUM_HEADS), jnp.float32),
        scratch_types=[
            pltpu.VMEM((n_chunks, _CHUNK), jnp.int32),
            pltpu.VMEM((b_per_w, NUM_HEADS), jnp.float32),
            pltpu.SemaphoreType.DMA,
        ],
        compiler_params=pltpu.CompilerParams(use_tc_tiling_on_sc=False),
    )
    def gate_kernel(idx_hbm, table_hbm, out_hbm, idx_v, rows_v, sem):
        wid = lax.axis_index("s") * _NC + lax.axis_index("c")
        pltpu.sync_copy(idx_hbm.at[pl.ds(wid * n_chunks, n_chunks)], idx_v)
        copies = [
            pltpu.async_copy(
                table_hbm.at[idx_v.at[j]],
                rows_v.at[pl.ds(j * _CHUNK, _CHUNK)],
                sem,
            )
            for j in range(n_chunks)
        ]
        for c in copies:
            c.wait()

        lanes = lax.iota(jnp.int32, 16)
        perms = [lanes ^ (1 << k) for k in range(4)]

        def body(i, carry):
            e = jnp.exp(rows_v[i, :])
            s = e
            for p in perms:
                s = s + s.at[p].get(mode="promise_in_bounds")
            rows_v[i, :] = e / s
            return carry

        lax.fori_loop(0, b_per_w, body, 0)

        pltpu.sync_copy(rows_v, out_hbm.at[pl.ds(wid * b_per_w, b_per_w)])

    return gate_kernel


def kernel(user_idx, logits):
    B = user_idx.shape[0]
    idx2d = user_idx.astype(jnp.int32).reshape(B // _CHUNK, _CHUNK)
    return _build(B)(idx2d, logits)


# speedup vs baseline: 1.6185x; 1.6185x over previous
import functools

import jax
import jax.numpy as jnp
from jax import lax
from jax.experimental import pallas as pl
from jax.experimental.pallas import tpu as pltpu
from jax.experimental.pallas import tpu_sc as plsc

NUM_HEADS = 16
_NC, _NS = 2, 16
_NW = _NC * _NS
_UNROLL = 8


@functools.lru_cache(maxsize=None)
def _build(B):
    b_per_w = B // _NW
    mesh = plsc.VectorSubcoreMesh(core_axis_name="c", subcore_axis_name="s")

    @functools.partial(
        pl.kernel,
        mesh=mesh,
        out_type=jax.ShapeDtypeStruct((B, NUM_HEADS), jnp.float32),
        scratch_types=[
            pltpu.VMEM((b_per_w,), jnp.int32),
            pltpu.VMEM((b_per_w, NUM_HEADS), jnp.float32),
            pltpu.SemaphoreType.DMA,
        ],
        compiler_params=pltpu.CompilerParams(use_tc_tiling_on_sc=True),
    )
    def gate_kernel(idx_hbm, table_hbm, out_hbm, idx_s, rows_v, sem):
        wid = lax.axis_index("s") * _NC + lax.axis_index("c")
        base = wid * b_per_w
        pltpu.sync_copy(idx_hbm.at[pl.ds(base, b_per_w)], idx_s)

        def fire(i, carry):
            iv = idx_s[pl.ds(i * 16, 16)]
            for u in range(16):
                r = iv[u]
                pltpu.async_copy(
                    table_hbm.at[pl.ds(r, 1)],
                    rows_v.at[pl.ds(i * 16 + u, 1)],
                    sem,
                )
            return carry

        lax.fori_loop(0, b_per_w // 16, fire, 0)
        pltpu.make_async_copy(
            out_hbm.at[pl.ds(base, b_per_w)], rows_v, sem
        ).wait()

        lanes = lax.iota(jnp.int32, 16)
        perms = [lanes ^ (1 << k) for k in range(4)]

        def body(i, carry):
            e = jnp.exp(rows_v[i, :])
            s = e
            for p in perms:
                s = s + s.at[p].get(mode="promise_in_bounds")
            rows_v[i, :] = e / s
            return carry

        lax.fori_loop(0, b_per_w, body, 0)

        pltpu.sync_copy(rows_v, out_hbm.at[pl.ds(base, b_per_w)])

    return gate_kernel


def kernel(user_idx, logits):
    B = user_idx.shape[0]
    return _build(B)(user_idx.astype(jnp.int32), logits)
